# baseline (speedup 1.0000x reference)
import functools

import jax
import jax.numpy as jnp
from jax import lax
from jax.experimental import pallas as pl
from jax.experimental.pallas import tpu as pltpu
from jax.experimental.pallas import tpu_sc as plsc

_NC = 2
_NS = 16
_NW = _NC * _NS
_CHUNK = 128


def _sc_gather(stu_id, exer_id, mean_table, cov_table, kd_table, ed_flat):
  B = stu_id.shape[0]
  KN = mean_table.shape[1]
  b_per_w = B // _NW
  n_chunks = b_per_w // _CHUNK

  stu_idx3 = stu_id.reshape(_NW, n_chunks, _CHUNK)
  exer_idx3 = exer_id.reshape(_NW, n_chunks, _CHUNK)

  mesh = plsc.VectorSubcoreMesh(
      core_axis_name="c", subcore_axis_name="s", num_cores=_NC,
      num_subcores=_NS)

  @functools.partial(
      pl.kernel,
      out_type=(
          jax.ShapeDtypeStruct((B, KN), jnp.float32),
          jax.ShapeDtypeStruct((B, KN), jnp.float32),
          jax.ShapeDtypeStruct((B, KN), jnp.float32),
          jax.ShapeDtypeStruct((B,), jnp.float32),
      ),
      mesh=mesh,
      compiler_params=pltpu.CompilerParams(use_tc_tiling_on_sc=True),
      scratch_types=[
          pltpu.VMEM((n_chunks, _CHUNK), jnp.int32),
          pltpu.VMEM((n_chunks, _CHUNK), jnp.int32),
          pltpu.VMEM((2, _CHUNK, KN), jnp.float32),
          pltpu.VMEM((2, _CHUNK, KN), jnp.float32),
          pltpu.VMEM((2, _CHUNK, KN), jnp.float32),
          pltpu.VMEM((2, _CHUNK), jnp.float32),
          pltpu.SemaphoreType.DMA,
          pltpu.SemaphoreType.DMA,
          pltpu.SemaphoreType.DMA,
          pltpu.SemaphoreType.DMA,
      ],
  )
  def gather_kernel(stu_idx_hbm, exer_idx_hbm, mean_hbm, cov_hbm, kd_hbm,
                    ed_hbm, mean_out, cov_out, kd_out, ed_out,
                    sidx_v, eidx_v, mean_v, cov_v, kd_v, ed_v,
                    gsem0, gsem1, wsem0, wsem1):
    wid = lax.axis_index("s") * _NC + lax.axis_index("c")
    base = wid * b_per_w
    pltpu.sync_copy(stu_idx_hbm.at[wid], sidx_v)
    pltpu.sync_copy(exer_idx_hbm.at[wid], eidx_v)
    gsems = (gsem0, gsem1)
    wsems = (wsem0, wsem1)

    def fire_gather(c):
      s = c % 2
      return [
          pltpu.async_copy(mean_hbm.at[sidx_v.at[c]], mean_v.at[s], gsems[s]),
          pltpu.async_copy(cov_hbm.at[sidx_v.at[c]], cov_v.at[s], gsems[s]),
          pltpu.async_copy(kd_hbm.at[eidx_v.at[c]], kd_v.at[s], gsems[s]),
          pltpu.async_copy(ed_hbm.at[eidx_v.at[c]], ed_v.at[s], gsems[s]),
      ]

    def fire_writeback(c):
      s = c % 2
      off = base + c * _CHUNK
      return [
          pltpu.async_copy(mean_v.at[s], mean_out.at[pl.ds(off, _CHUNK)],
                           wsems[s]),
          pltpu.async_copy(cov_v.at[s], cov_out.at[pl.ds(off, _CHUNK)],
                           wsems[s]),
          pltpu.async_copy(kd_v.at[s], kd_out.at[pl.ds(off, _CHUNK)],
                           wsems[s]),
          pltpu.async_copy(ed_v.at[s], ed_out.at[pl.ds(off, _CHUNK)],
                           wsems[s]),
      ]

    gat = {0: fire_gather(0)}
    wbs = {}
    for c in range(n_chunks):
      if c + 1 < n_chunks:
        if c - 1 >= 0:
          for d in wbs[c - 1]:
            d.wait()
        gat[c + 1] = fire_gather(c + 1)
      for d in gat[c]:
        d.wait()
      wbs[c] = fire_writeback(c)
    for d in wbs[n_chunks - 2] + wbs[n_chunks - 1]:
      d.wait()

  return gather_kernel(stu_idx3, exer_idx3, mean_table, cov_table, kd_table,
                       ed_flat)


def _tc_mlp(stu_mean, kd_raw, ed_raw, kn_id, w1q, b1q, w2q, b2q, w3q, b3q):
  B, KN = stu_mean.shape
  L1 = w1q.shape[0]
  L2 = w2q.shape[0]
  BB = 2048
  grid = (B // BB,)

  dn = (((1,), (1,)), ((), ()))

  def body(mean_ref, kd_ref, ed_ref, kn_ref, w1_ref, b1_ref, w2_ref, b2_ref,
           w3_ref, b3_ref, out_ref):
    bf = jnp.bfloat16
    t_m = jnp.tanh((0.5 * mean_ref[...]).astype(bf))
    t_k = jnp.tanh((0.5 * kd_ref[...]).astype(bf))
    c_row = 2.5 * (jnp.tanh(0.5 * ed_ref[...]) + 1.0)
    c = jnp.reshape(c_row, (c_row.shape[1], 1))
    x = (c * (t_m - t_k).astype(jnp.float32) * kn_ref[...]).astype(bf)
    t1 = jnp.tanh(
        (lax.dot_general(x, w1_ref[...], dn,
                         preferred_element_type=jnp.float32)
         + b1_ref[...]).astype(bf))
    t2 = jnp.tanh(
        (lax.dot_general(t1, w2_ref[...], dn,
                         preferred_element_type=jnp.float32)
         + b2_ref[...]).astype(bf))
    ot = lax.dot_general(w3_ref[...], t2, dn,
                         preferred_element_type=jnp.float32)
    out_ref[...] = 0.5 * jnp.tanh(ot + b3_ref[...]) + 0.5

  return pl.pallas_call(
      body,
      grid=grid,
      in_specs=[
          pl.BlockSpec((BB, KN), lambda i: (i, 0)),
          pl.BlockSpec((BB, KN), lambda i: (i, 0)),
          pl.BlockSpec((1, BB), lambda i: (0, i)),
          pl.BlockSpec((BB, KN), lambda i: (i, 0)),
          pl.BlockSpec((L1, KN), lambda i: (0, 0)),
          pl.BlockSpec((1, L1), lambda i: (0, 0)),
          pl.BlockSpec((L2, L1), lambda i: (0, 0)),
          pl.BlockSpec((1, L2), lambda i: (0, 0)),
          pl.BlockSpec((1, L2), lambda i: (0, 0)),
          pl.BlockSpec((1, 1), lambda i: (0, 0)),
      ],
      out_specs=pl.BlockSpec((1, BB), lambda i: (0, i)),
      out_shape=jax.ShapeDtypeStruct((1, B), jnp.float32),
  )(stu_mean, kd_raw, ed_raw, kn_id, w1q, b1q, w2q, b2q, w3q, b3q)


def kernel(stu_id, exer_id, kn_id, d_type, mean_table, cov_table, kd_table,
           ed_table, W1, b1, W2, b2, W3, b3):
  bf = jnp.bfloat16
  w1q = (0.5 * W1).astype(bf)
  b1q = (0.5 * b1).reshape(1, -1)
  w2q = (0.25 * W2).astype(bf)
  b2q = (0.5 * (b2 + 0.5 * W2.sum(axis=1))).reshape(1, -1)
  w3q = (0.25 * W3).astype(bf)
  b3q = (0.5 * (b3 + 0.5 * W3.sum(axis=1))).reshape(1, -1)
  ed_flat = ed_table.reshape(-1)

  B = stu_id.shape[0]
  H = B // 2
  outs, means, covs = [], [], []
  for h in range(2):
    sl = slice(h * H, (h + 1) * H)
    m_h, cv_h, kd_h, ed_h = _sc_gather(
        stu_id[sl], exer_id[sl], mean_table, cov_table, kd_table, ed_flat)
    out_h = _tc_mlp(m_h, kd_h, ed_h.reshape(1, -1), kn_id[sl],
                    w1q, b1q, w2q, b2q, w3q, b3q)
    outs.append(out_h)
    means.append(m_h)
    covs.append(cv_h)
  out_row = jnp.concatenate(outs, axis=1)
  stu_mean = jnp.concatenate(means, axis=0)
  log_cov = jnp.concatenate(covs, axis=0)
  return (out_row.reshape(-1, 1), stu_mean, log_cov)


# speedup vs baseline: 1.2281x; 1.2281x over previous
import functools

import jax
import jax.numpy as jnp
from jax import lax
from jax.experimental import pallas as pl
from jax.experimental.pallas import tpu as pltpu
from jax.experimental.pallas import tpu_sc as plsc

_NC = 2
_NS = 16
_NW = _NC * _NS
_CHUNK = 128


def _sc_gather(stu_id, exer_id, mean_table, cov_table, kd_table, ed_flat):
  B = stu_id.shape[0]
  KN = mean_table.shape[1]
  b_per_w = B // _NW
  n_chunks = b_per_w // _CHUNK

  stu_idx3 = stu_id.reshape(_NW, n_chunks, _CHUNK)
  exer_idx3 = exer_id.reshape(_NW, n_chunks, _CHUNK)

  mesh = plsc.VectorSubcoreMesh(
      core_axis_name="c", subcore_axis_name="s", num_cores=_NC,
      num_subcores=_NS)

  @functools.partial(
      pl.kernel,
      out_type=(
          jax.ShapeDtypeStruct((B, KN), jnp.float32),
          jax.ShapeDtypeStruct((B, KN), jnp.float32),
          jax.ShapeDtypeStruct((B, KN), jnp.float32),
          jax.ShapeDtypeStruct((B,), jnp.float32),
      ),
      mesh=mesh,
      compiler_params=pltpu.CompilerParams(use_tc_tiling_on_sc=True),
      scratch_types=[
          pltpu.VMEM((n_chunks, _CHUNK), jnp.int32),
          pltpu.VMEM((n_chunks, _CHUNK), jnp.int32),
          pltpu.VMEM((2, _CHUNK, KN), jnp.float32),
          pltpu.VMEM((2, _CHUNK, KN), jnp.float32),
          pltpu.VMEM((2, _CHUNK, KN), jnp.float32),
          pltpu.VMEM((2, _CHUNK), jnp.float32),
          pltpu.SemaphoreType.DMA,
          pltpu.SemaphoreType.DMA,
          pltpu.SemaphoreType.DMA,
          pltpu.SemaphoreType.DMA,
      ],
  )
  def gather_kernel(stu_idx_hbm, exer_idx_hbm, mean_hbm, cov_hbm, kd_hbm,
                    ed_hbm, mean_out, cov_out, kd_out, ed_out,
                    sidx_v, eidx_v, mean_v, cov_v, kd_v, ed_v,
                    gsem0, gsem1, wsem0, wsem1):
    wid = lax.axis_index("s") * _NC + lax.axis_index("c")
    base = wid * b_per_w
    pltpu.sync_copy(stu_idx_hbm.at[wid], sidx_v)
    pltpu.sync_copy(exer_idx_hbm.at[wid], eidx_v)
    gsems = (gsem0, gsem1)
    wsems = (wsem0, wsem1)

    def fire_gather(c):
      s = c % 2
      return [
          pltpu.async_copy(mean_hbm.at[sidx_v.at[c]], mean_v.at[s], gsems[s]),
          pltpu.async_copy(cov_hbm.at[sidx_v.at[c]], cov_v.at[s], gsems[s]),
          pltpu.async_copy(kd_hbm.at[eidx_v.at[c]], kd_v.at[s], gsems[s]),
          pltpu.async_copy(ed_hbm.at[eidx_v.at[c]], ed_v.at[s], gsems[s]),
      ]

    def fire_writeback(c):
      s = c % 2
      off = base + c * _CHUNK
      return [
          pltpu.async_copy(mean_v.at[s], mean_out.at[pl.ds(off, _CHUNK)],
                           wsems[s]),
          pltpu.async_copy(cov_v.at[s], cov_out.at[pl.ds(off, _CHUNK)],
                           wsems[s]),
          pltpu.async_copy(kd_v.at[s], kd_out.at[pl.ds(off, _CHUNK)],
                           wsems[s]),
          pltpu.async_copy(ed_v.at[s], ed_out.at[pl.ds(off, _CHUNK)],
                           wsems[s]),
      ]

    gat = {0: fire_gather(0)}
    wbs = {}
    for c in range(n_chunks):
      if c + 1 < n_chunks:
        if c - 1 >= 0:
          for d in wbs[c - 1]:
            d.wait()
        gat[c + 1] = fire_gather(c + 1)
      for d in gat[c]:
        d.wait()
      wbs[c] = fire_writeback(c)
    for d in wbs[n_chunks - 2] + wbs[n_chunks - 1]:
      d.wait()

  return gather_kernel(stu_idx3, exer_idx3, mean_table, cov_table, kd_table,
                       ed_flat)


def _tc_mlp(stu_mean, kd_raw, ed_raw, kn_id, w1q, b1q, w2q, b2q, w3q, b3q):
  B, KN = stu_mean.shape
  L1 = w1q.shape[0]
  L2 = w2q.shape[0]
  BB = 1024
  grid = (B // BB,)

  dn = (((1,), (1,)), ((), ()))

  def body(mean_ref, kd_ref, ed_ref, kn_ref, w1_ref, b1_ref, w2_ref, b2_ref,
           w3_ref, b3_ref, out_ref):
    bf = jnp.bfloat16
    t_m = jnp.tanh((0.5 * mean_ref[...]).astype(bf))
    t_k = jnp.tanh((0.5 * kd_ref[...]).astype(bf))
    c_row = 2.5 * (jnp.tanh(0.5 * ed_ref[...]) + 1.0)
    c = jnp.reshape(c_row, (c_row.shape[1], 1))
    x = (c * (t_m - t_k).astype(jnp.float32) * kn_ref[...]).astype(bf)
    t1 = jnp.tanh(
        (lax.dot_general(x, w1_ref[...], dn,
                         preferred_element_type=jnp.float32)
         + b1_ref[...]).astype(bf))
    t2 = jnp.tanh(
        (lax.dot_general(t1, w2_ref[...], dn,
                         preferred_element_type=jnp.float32)
         + b2_ref[...]).astype(bf))
    ot = lax.dot_general(w3_ref[...], t2, dn,
                         preferred_element_type=jnp.float32)
    out_ref[...] = 0.5 * jnp.tanh(ot + b3_ref[...]) + 0.5

  return pl.pallas_call(
      body,
      grid=grid,
      in_specs=[
          pl.BlockSpec((BB, KN), lambda i: (i, 0)),
          pl.BlockSpec((BB, KN), lambda i: (i, 0)),
          pl.BlockSpec((1, BB), lambda i: (0, i)),
          pl.BlockSpec((BB, KN), lambda i: (i, 0)),
          pl.BlockSpec((L1, KN), lambda i: (0, 0)),
          pl.BlockSpec((1, L1), lambda i: (0, 0)),
          pl.BlockSpec((L2, L1), lambda i: (0, 0)),
          pl.BlockSpec((1, L2), lambda i: (0, 0)),
          pl.BlockSpec((1, L2), lambda i: (0, 0)),
          pl.BlockSpec((1, 1), lambda i: (0, 0)),
      ],
      out_specs=pl.BlockSpec((1, BB), lambda i: (0, i)),
      out_shape=jax.ShapeDtypeStruct((1, B), jnp.float32),
  )(stu_mean, kd_raw, ed_raw, kn_id, w1q, b1q, w2q, b2q, w3q, b3q)


def kernel(stu_id, exer_id, kn_id, d_type, mean_table, cov_table, kd_table,
           ed_table, W1, b1, W2, b2, W3, b3):
  bf = jnp.bfloat16
  w1q = (0.5 * W1).astype(bf)
  b1q = (0.5 * b1).reshape(1, -1)
  w2q = (0.25 * W2).astype(bf)
  b2q = (0.5 * (b2 + 0.5 * W2.sum(axis=1))).reshape(1, -1)
  w3q = (0.25 * W3).astype(bf)
  b3q = (0.5 * (b3 + 0.5 * W3.sum(axis=1))).reshape(1, -1)
  stu_mean, log_cov, kd_raw, ed_raw = _sc_gather(
      stu_id, exer_id, mean_table, cov_table, kd_table, ed_table.reshape(-1))
  out_row = _tc_mlp(stu_mean, kd_raw, ed_raw.reshape(1, -1), kn_id,
                    w1q, b1q, w2q, b2q, w3q, b3q)
  return (out_row.reshape(-1, 1), stu_mean, log_cov)


# speedup vs baseline: 1.3300x; 1.0831x over previous
import functools

import jax
import jax.numpy as jnp
from jax import lax
from jax.experimental import pallas as pl
from jax.experimental.pallas import tpu as pltpu
from jax.experimental.pallas import tpu_sc as plsc

_NC = 2
_NS = 16
_NW = _NC * _NS
_CHUNK = 128


def _sc_gather(stu_id, exer_id, mean_table, cov_table, kd_table, ed_flat):
  B = stu_id.shape[0]
  KN = mean_table.shape[1]
  b_per_w = B // _NW
  n_chunks = b_per_w // _CHUNK

  stu_idx3 = stu_id.reshape(_NW, n_chunks, _CHUNK)
  exer_idx3 = exer_id.reshape(_NW, n_chunks, _CHUNK)

  mesh = plsc.VectorSubcoreMesh(
      core_axis_name="c", subcore_axis_name="s", num_cores=_NC,
      num_subcores=_NS)

  @functools.partial(
      pl.kernel,
      out_type=(
          jax.ShapeDtypeStruct((B, KN), jnp.float32),
          jax.ShapeDtypeStruct((B, KN), jnp.float32),
          jax.ShapeDtypeStruct((B, KN), jnp.float32),
          jax.ShapeDtypeStruct((B,), jnp.float32),
      ),
      mesh=mesh,
      compiler_params=pltpu.CompilerParams(use_tc_tiling_on_sc=True),
      scratch_types=[
          pltpu.VMEM((n_chunks, _CHUNK), jnp.int32),
          pltpu.VMEM((n_chunks, _CHUNK), jnp.int32),
          pltpu.VMEM((2, _CHUNK, KN), jnp.float32),
          pltpu.VMEM((2, _CHUNK, KN), jnp.float32),
          pltpu.VMEM((2, _CHUNK, KN), jnp.float32),
          pltpu.VMEM((2, _CHUNK), jnp.float32),
          pltpu.SemaphoreType.DMA,
          pltpu.SemaphoreType.DMA,
          pltpu.SemaphoreType.DMA,
          pltpu.SemaphoreType.DMA,
      ],
  )
  def gather_kernel(stu_idx_hbm, exer_idx_hbm, mean_hbm, cov_hbm, kd_hbm,
                    ed_hbm, mean_out, cov_out, kd_out, ed_out,
                    sidx_v, eidx_v, mean_v, cov_v, kd_v, ed_v,
                    gsem0, gsem1, wsem0, wsem1):
    wid = lax.axis_index("s") * _NC + lax.axis_index("c")
    base = wid * b_per_w
    pltpu.sync_copy(stu_idx_hbm.at[wid], sidx_v)
    pltpu.sync_copy(exer_idx_hbm.at[wid], eidx_v)
    gsems = (gsem0, gsem1)
    wsems = (wsem0, wsem1)

    def fire_gather(c):
      s = c % 2
      return [
          pltpu.async_copy(mean_hbm.at[sidx_v.at[c]], mean_v.at[s], gsems[s]),
          pltpu.async_copy(cov_hbm.at[sidx_v.at[c]], cov_v.at[s], gsems[s]),
          pltpu.async_copy(kd_hbm.at[eidx_v.at[c]], kd_v.at[s], gsems[s]),
          pltpu.async_copy(ed_hbm.at[eidx_v.at[c]], ed_v.at[s], gsems[s]),
      ]

    def fire_writeback(c):
      s = c % 2
      off = base + c * _CHUNK
      return [
          pltpu.async_copy(mean_v.at[s], mean_out.at[pl.ds(off, _CHUNK)],
                           wsems[s]),
          pltpu.async_copy(cov_v.at[s], cov_out.at[pl.ds(off, _CHUNK)],
                           wsems[s]),
          pltpu.async_copy(kd_v.at[s], kd_out.at[pl.ds(off, _CHUNK)],
                           wsems[s]),
          pltpu.async_copy(ed_v.at[s], ed_out.at[pl.ds(off, _CHUNK)],
                           wsems[s]),
      ]

    gat = {0: fire_gather(0)}
    wbs = {}
    for c in range(n_chunks):
      if c + 1 < n_chunks:
        if c - 1 >= 0:
          for d in wbs[c - 1]:
            d.wait()
        gat[c + 1] = fire_gather(c + 1)
      for d in gat[c]:
        d.wait()
      wbs[c] = fire_writeback(c)
    for d in wbs[n_chunks - 2] + wbs[n_chunks - 1]:
      d.wait()

  return gather_kernel(stu_idx3, exer_idx3, mean_table, cov_table, kd_table,
                       ed_flat)


def _tc_mlp(stu_mean, kd_raw, ed_raw, kn_id, w1q, b1q, w2q, b2q, w3q, b3q):
  B, KN = stu_mean.shape
  L1 = w1q.shape[0]
  L2 = w2q.shape[0]
  BB = 4096
  grid = (B // BB,)

  dn = (((1,), (1,)), ((), ()))

  def body(mean_ref, kd_ref, ed_ref, kn_ref, w1_ref, b1_ref, w2_ref, b2_ref,
           w3_ref, b3_ref, out_ref):
    bf = jnp.bfloat16
    t_m = jnp.tanh((0.5 * mean_ref[...]).astype(bf))
    t_k = jnp.tanh((0.5 * kd_ref[...]).astype(bf))
    c_row = 2.5 * (jnp.tanh(0.5 * ed_ref[...]) + 1.0)
    c = jnp.reshape(c_row, (c_row.shape[1], 1))
    x = (c * (t_m - t_k).astype(jnp.float32) * kn_ref[...]).astype(bf)
    t1 = jnp.tanh(
        (lax.dot_general(x, w1_ref[...], dn,
                         preferred_element_type=jnp.float32)
         + b1_ref[...]).astype(bf))
    t2 = jnp.tanh(
        (lax.dot_general(t1, w2_ref[...], dn,
                         preferred_element_type=jnp.float32)
         + b2_ref[...]).astype(bf))
    ot = lax.dot_general(w3_ref[...], t2, dn,
                         preferred_element_type=jnp.float32)
    out_ref[...] = 0.5 * jnp.tanh(ot + b3_ref[...]) + 0.5

  return pl.pallas_call(
      body,
      grid=grid,
      in_specs=[
          pl.BlockSpec((BB, KN), lambda i: (i, 0)),
          pl.BlockSpec((BB, KN), lambda i: (i, 0)),
          pl.BlockSpec((1, BB), lambda i: (0, i)),
          pl.BlockSpec((BB, KN), lambda i: (i, 0)),
          pl.BlockSpec((L1, KN), lambda i: (0, 0)),
          pl.BlockSpec((1, L1), lambda i: (0, 0)),
          pl.BlockSpec((L2, L1), lambda i: (0, 0)),
          pl.BlockSpec((1, L2), lambda i: (0, 0)),
          pl.BlockSpec((1, L2), lambda i: (0, 0)),
          pl.BlockSpec((1, 1), lambda i: (0, 0)),
      ],
      out_specs=pl.BlockSpec((1, BB), lambda i: (0, i)),
      out_shape=jax.ShapeDtypeStruct((1, B), jnp.float32),
  )(stu_mean, kd_raw, ed_raw, kn_id, w1q, b1q, w2q, b2q, w3q, b3q)


def kernel(stu_id, exer_id, kn_id, d_type, mean_table, cov_table, kd_table,
           ed_table, W1, b1, W2, b2, W3, b3):
  bf = jnp.bfloat16
  w1q = (0.5 * W1).astype(bf)
  b1q = (0.5 * b1).reshape(1, -1)
  w2q = (0.25 * W2).astype(bf)
  b2q = (0.5 * (b2 + 0.5 * W2.sum(axis=1))).reshape(1, -1)
  w3q = (0.25 * W3).astype(bf)
  b3q = (0.5 * (b3 + 0.5 * W3.sum(axis=1))).reshape(1, -1)
  stu_mean, log_cov, kd_raw, ed_raw = _sc_gather(
      stu_id, exer_id, mean_table, cov_table, kd_table, ed_table.reshape(-1))
  out_row = _tc_mlp(stu_mean, kd_raw, ed_raw.reshape(1, -1), kn_id,
                    w1q, b1q, w2q, b2q, w3q, b3q)
  return (out_row.reshape(-1, 1), stu_mean, log_cov)
